# baseline (device time: 61580 ns/iter reference)
import jax
import jax.numpy as jnp
from jax import lax
from jax.experimental import pallas as pl
from jax.experimental.pallas import tpu as pltpu

N_DEV = 8


def kernel(x, dy):
    m, d_in = x.shape
    _, f = dy.shape
    rows_out = d_in // N_DEV

    def body(x_ref, dy_ref, out_ref, acc_ref, comm_ref, send_sems, recv_sems):
        my = lax.axis_index("i")
        left = lax.rem(my + N_DEV - 1, N_DEV)
        right = lax.rem(my + 1, N_DEV)

        barrier_sem = pltpu.get_barrier_semaphore()
        for nbr in (left, right):
            pl.semaphore_signal(
                barrier_sem, inc=1,
                device_id=(nbr,), device_id_type=pl.DeviceIdType.MESH,
            )
        pl.semaphore_wait(barrier_sem, 2)

        acc_ref[:, :] = lax.dot_general(
            x_ref[:, :], dy_ref[:, :],
            dimension_numbers=(((0,), (0,)), ((), ())),
            preferred_element_type=jnp.float32,
        )

        c0 = lax.rem(my + N_DEV - 1, N_DEV)
        comm_ref[0, :, :] = acc_ref[pl.ds(c0 * rows_out, rows_out), :]
        for s in range(N_DEV - 1):
            send_slot = s % 2
            recv_slot = (s + 1) % 2
            if s > 0:
                c_s = lax.rem(my + N_DEV - 1 - s, N_DEV)
                comm_ref[send_slot, :, :] = (
                    comm_ref[send_slot, :, :]
                    + acc_ref[pl.ds(c_s * rows_out, rows_out), :]
                )
            rdma = pltpu.make_async_remote_copy(
                src_ref=comm_ref.at[send_slot],
                dst_ref=comm_ref.at[recv_slot],
                send_sem=send_sems.at[send_slot],
                recv_sem=recv_sems.at[recv_slot],
                device_id=(right,),
                device_id_type=pl.DeviceIdType.MESH,
            )
            rdma.start()
            rdma.wait()

        out_ref[:, :] = (
            comm_ref[(N_DEV - 1) % 2, :, :]
            + acc_ref[pl.ds(my * rows_out, rows_out), :]
        )

    return pl.pallas_call(
        body,
        out_shape=jax.ShapeDtypeStruct((rows_out, f), jnp.float32),
        in_specs=[
            pl.BlockSpec(memory_space=pltpu.VMEM),
            pl.BlockSpec(memory_space=pltpu.VMEM),
        ],
        out_specs=pl.BlockSpec(memory_space=pltpu.VMEM),
        scratch_shapes=[
            pltpu.VMEM((d_in, f), jnp.float32),
            pltpu.VMEM((2, rows_out, f), jnp.float32),
            pltpu.SemaphoreType.DMA((2,)),
            pltpu.SemaphoreType.DMA((2,)),
        ],
        compiler_params=pltpu.CompilerParams(collective_id=0),
    )(x, dy)


# device time: 44805 ns/iter; 1.3744x vs baseline; 1.3744x over previous
import jax
import jax.numpy as jnp
from jax import lax
from jax.experimental import pallas as pl
from jax.experimental.pallas import tpu as pltpu

N_DEV = 8


def kernel(x, dy):
    m, d_in = x.shape
    _, f = dy.shape
    rows_out = d_in // N_DEV
    fh = f // 2

    def body(x_ref, dy_ref, out_ref, acc_ref,
             cw_ref, ccw_ref, cw_send_sems, cw_recv_sems,
             ccw_send_sems, ccw_recv_sems):
        my = lax.axis_index("i")
        left = lax.rem(my + N_DEV - 1, N_DEV)
        right = lax.rem(my + 1, N_DEV)

        barrier_sem = pltpu.get_barrier_semaphore()
        for nbr in (left, right):
            pl.semaphore_signal(
                barrier_sem, inc=1,
                device_id=(nbr,), device_id_type=pl.DeviceIdType.MESH,
            )
        pl.semaphore_wait(barrier_sem, 2)

        acc_ref[:, :] = lax.dot_general(
            x_ref[:, :], dy_ref[:, :],
            dimension_numbers=(((0,), (0,)), ((), ())),
            preferred_element_type=jnp.float32,
        )

        c_cw0 = lax.rem(my + N_DEV - 1, N_DEV)
        c_ccw0 = lax.rem(my + 1, N_DEV)
        cw_ref[0, :, :] = acc_ref[pl.ds(c_cw0 * rows_out, rows_out), :fh]
        ccw_ref[0, :, :] = acc_ref[pl.ds(c_ccw0 * rows_out, rows_out), fh:]

        for s in range(N_DEV - 1):
            send_slot = s % 2
            recv_slot = (s + 1) % 2
            if s > 0:
                c_cw = lax.rem(my + N_DEV - 1 - s, N_DEV)
                c_ccw = lax.rem(my + 1 + s, N_DEV)
                cw_ref[send_slot, :, :] = (
                    cw_ref[send_slot, :, :]
                    + acc_ref[pl.ds(c_cw * rows_out, rows_out), :fh]
                )
                ccw_ref[send_slot, :, :] = (
                    ccw_ref[send_slot, :, :]
                    + acc_ref[pl.ds(c_ccw * rows_out, rows_out), fh:]
                )
            cw = pltpu.make_async_remote_copy(
                src_ref=cw_ref.at[send_slot],
                dst_ref=cw_ref.at[recv_slot],
                send_sem=cw_send_sems.at[send_slot],
                recv_sem=cw_recv_sems.at[recv_slot],
                device_id=(right,),
                device_id_type=pl.DeviceIdType.MESH,
            )
            ccw = pltpu.make_async_remote_copy(
                src_ref=ccw_ref.at[send_slot],
                dst_ref=ccw_ref.at[recv_slot],
                send_sem=ccw_send_sems.at[send_slot],
                recv_sem=ccw_recv_sems.at[recv_slot],
                device_id=(left,),
                device_id_type=pl.DeviceIdType.MESH,
            )
            cw.start()
            ccw.start()
            cw.wait()
            ccw.wait()

        last = (N_DEV - 1) % 2
        out_ref[:, :fh] = (
            cw_ref[last, :, :] + acc_ref[pl.ds(my * rows_out, rows_out), :fh]
        )
        out_ref[:, fh:] = (
            ccw_ref[last, :, :] + acc_ref[pl.ds(my * rows_out, rows_out), fh:]
        )

    return pl.pallas_call(
        body,
        out_shape=jax.ShapeDtypeStruct((rows_out, f), jnp.float32),
        in_specs=[
            pl.BlockSpec(memory_space=pltpu.VMEM),
            pl.BlockSpec(memory_space=pltpu.VMEM),
        ],
        out_specs=pl.BlockSpec(memory_space=pltpu.VMEM),
        scratch_shapes=[
            pltpu.VMEM((d_in, f), jnp.float32),
            pltpu.VMEM((2, rows_out, fh), jnp.float32),
            pltpu.VMEM((2, rows_out, fh), jnp.float32),
            pltpu.SemaphoreType.DMA((2,)),
            pltpu.SemaphoreType.DMA((2,)),
            pltpu.SemaphoreType.DMA((2,)),
            pltpu.SemaphoreType.DMA((2,)),
        ],
        compiler_params=pltpu.CompilerParams(collective_id=0),
    )(x, dy)


# device time: 33791 ns/iter; 1.8224x vs baseline; 1.3259x over previous
import jax
import jax.numpy as jnp
from jax import lax
from jax.experimental import pallas as pl
from jax.experimental.pallas import tpu as pltpu

N_DEV = 8
HOPS = N_DEV - 1
S = 2


def kernel(x, dy):
    m, d_in = x.shape
    _, f = dy.shape
    rows = d_in // N_DEV
    n_streams = 2 * S
    fq = f // n_streams

    stream_dirs = []
    for k in range(S):
        stream_dirs.append(True)
        stream_dirs.append(False)

    def body(x_ref, dy_ref, out_ref, acc_ref, *rest):
        comm = rest[:n_streams]
        send_sems = rest[n_streams:2 * n_streams]
        recv_sems = rest[2 * n_streams:3 * n_streams]

        my = lax.axis_index("i")
        left = lax.rem(my + N_DEV - 1, N_DEV)
        right = lax.rem(my + 1, N_DEV)

        barrier_sem = pltpu.get_barrier_semaphore()
        for nbr in (left, right):
            pl.semaphore_signal(
                barrier_sem, inc=1,
                device_id=(nbr,), device_id_type=pl.DeviceIdType.MESH,
            )
        pl.semaphore_wait(barrier_sem, 2)

        acc_ref[:, :] = lax.dot_general(
            x_ref[:, :], dy_ref[:, :],
            dimension_numbers=(((0,), (0,)), ((), ())),
            preferred_element_type=jnp.float32,
        )

        def chunk_at(st, s):
            if stream_dirs[st]:
                return lax.rem(my + N_DEV - 1 - s, N_DEV)
            return lax.rem(my + 1 + s, N_DEV)

        def acc_block(st, s):
            return acc_ref[pl.ds(chunk_at(st, s) * rows, rows),
                           pl.ds(st * fq, fq)]

        def make_rdma(st, s):
            return pltpu.make_async_remote_copy(
                src_ref=comm[st].at[s],
                dst_ref=comm[st].at[s + 1],
                send_sem=send_sems[st].at[s],
                recv_sem=recv_sems[st].at[s],
                device_id=(right if stream_dirs[st] else left,),
                device_id_type=pl.DeviceIdType.MESH,
            )

        rdmas = {}

        for st in range(n_streams):
            comm[st][0, :, :] = acc_block(st, 0)
            rdmas[(st, 0)] = make_rdma(st, 0)
            rdmas[(st, 0)].start()

        for s in range(1, HOPS):
            for st in range(n_streams):
                rdmas[(st, s - 1)].wait_recv()
                comm[st][s, :, :] = comm[st][s, :, :] + acc_block(st, s)
                rdmas[(st, s)] = make_rdma(st, s)
                rdmas[(st, s)].start()

        for st in range(n_streams):
            rdmas[(st, HOPS - 1)].wait_recv()
            out_ref[:, pl.ds(st * fq, fq)] = (
                comm[st][HOPS, :, :]
                + acc_ref[pl.ds(my * rows, rows), pl.ds(st * fq, fq)]
            )

        for st in range(n_streams):
            for s in range(HOPS):
                rdmas[(st, s)].wait_send()

    return pl.pallas_call(
        body,
        out_shape=jax.ShapeDtypeStruct((rows, f), jnp.float32),
        in_specs=[
            pl.BlockSpec(memory_space=pltpu.VMEM),
            pl.BlockSpec(memory_space=pltpu.VMEM),
        ],
        out_specs=pl.BlockSpec(memory_space=pltpu.VMEM),
        scratch_shapes=(
            [pltpu.VMEM((d_in, f), jnp.float32)]
            + [pltpu.VMEM((HOPS + 1, rows, fq), jnp.float32)
               for _ in range(n_streams)]
            + [pltpu.SemaphoreType.DMA((HOPS,))
               for _ in range(n_streams)]
            + [pltpu.SemaphoreType.DMA((HOPS,))
               for _ in range(n_streams)]
        ),
        compiler_params=pltpu.CompilerParams(collective_id=0),
    )(x, dy)


# device time: 32587 ns/iter; 1.8897x vs baseline; 1.0369x over previous
import jax
import jax.numpy as jnp
from jax import lax
from jax.experimental import pallas as pl
from jax.experimental.pallas import tpu as pltpu

N_DEV = 8
HOPS = N_DEV - 1
S = 4


def kernel(x, dy):
    m, d_in = x.shape
    _, f = dy.shape
    rows = d_in // N_DEV
    n_streams = 2 * S
    fq = f // n_streams

    stream_dirs = []
    for k in range(S):
        stream_dirs.append(True)
        stream_dirs.append(False)

    def body(x_ref, dy_ref, out_ref, acc_ref, *rest):
        comm = rest[:n_streams]
        send_sems = rest[n_streams:2 * n_streams]
        recv_sems = rest[2 * n_streams:3 * n_streams]

        my = lax.axis_index("i")
        left = lax.rem(my + N_DEV - 1, N_DEV)
        right = lax.rem(my + 1, N_DEV)

        barrier_sem = pltpu.get_barrier_semaphore()
        for nbr in (left, right):
            pl.semaphore_signal(
                barrier_sem, inc=1,
                device_id=(nbr,), device_id_type=pl.DeviceIdType.MESH,
            )
        pl.semaphore_wait(barrier_sem, 2)

        acc_ref[:, :] = lax.dot_general(
            x_ref[:, :], dy_ref[:, :],
            dimension_numbers=(((0,), (0,)), ((), ())),
            preferred_element_type=jnp.float32,
        )

        def chunk_at(st, s):
            if stream_dirs[st]:
                return lax.rem(my + N_DEV - 1 - s, N_DEV)
            return lax.rem(my + 1 + s, N_DEV)

        def acc_block(st, s):
            return acc_ref[pl.ds(chunk_at(st, s) * rows, rows),
                           pl.ds(st * fq, fq)]

        def make_rdma(st, s):
            return pltpu.make_async_remote_copy(
                src_ref=comm[st].at[s],
                dst_ref=comm[st].at[s + 1],
                send_sem=send_sems[st].at[s],
                recv_sem=recv_sems[st].at[s],
                device_id=(right if stream_dirs[st] else left,),
                device_id_type=pl.DeviceIdType.MESH,
            )

        rdmas = {}

        for st in range(n_streams):
            comm[st][0, :, :] = acc_block(st, 0)
            rdmas[(st, 0)] = make_rdma(st, 0)
            rdmas[(st, 0)].start()

        for s in range(1, HOPS):
            for st in range(n_streams):
                rdmas[(st, s - 1)].wait_recv()
                comm[st][s, :, :] = comm[st][s, :, :] + acc_block(st, s)
                rdmas[(st, s)] = make_rdma(st, s)
                rdmas[(st, s)].start()

        for st in range(n_streams):
            rdmas[(st, HOPS - 1)].wait_recv()
            out_ref[:, pl.ds(st * fq, fq)] = (
                comm[st][HOPS, :, :]
                + acc_ref[pl.ds(my * rows, rows), pl.ds(st * fq, fq)]
            )

        for st in range(n_streams):
            for s in range(HOPS):
                rdmas[(st, s)].wait_send()

    return pl.pallas_call(
        body,
        out_shape=jax.ShapeDtypeStruct((rows, f), jnp.float32),
        in_specs=[
            pl.BlockSpec(memory_space=pltpu.VMEM),
            pl.BlockSpec(memory_space=pltpu.VMEM),
        ],
        out_specs=pl.BlockSpec(memory_space=pltpu.VMEM),
        scratch_shapes=(
            [pltpu.VMEM((d_in, f), jnp.float32)]
            + [pltpu.VMEM((HOPS + 1, rows, fq), jnp.float32)
               for _ in range(n_streams)]
            + [pltpu.SemaphoreType.DMA((HOPS,))
               for _ in range(n_streams)]
            + [pltpu.SemaphoreType.DMA((HOPS,))
               for _ in range(n_streams)]
        ),
        compiler_params=pltpu.CompilerParams(collective_id=0),
    )(x, dy)
